# baseline (speedup 1.0000x reference)
import functools

import jax
import jax.numpy as jnp
from jax import lax
from jax.experimental import pallas as pl
from jax.experimental.pallas import tpu as pltpu
from jax.experimental.pallas import tpu_sc as plsc

_NUM_CONTEXTS = 32
_NUM_GENS = 96
_MAX_DRAFT = 3
_ROWS = _NUM_CONTEXTS + _NUM_GENS * (_MAX_DRAFT + 1)
_VOCAB = 100000

_TC_ROWS = 288
_SC_ROWS = _ROWS - _TC_ROWS
_SC_GROUPS = _SC_ROWS // 8
_SC_GEN0 = (_TC_ROWS - _NUM_CONTEXTS) // 4

_VB = 16384
_NB = -(-_VOCAB // _VB)
_TAIL = _VOCAB - (_NB - 1) * _VB


def _tc_argmax_body(x_ref, tt_ref, val_ref, m_scr, a_scr):
    j = pl.program_id(0)

    def _reduce(x):
        col = jax.lax.broadcasted_iota(jnp.int32, (_TC_ROWS, _VB), 1)
        lmax = jnp.max(x, axis=1, keepdims=True)
        larg = jnp.min(jnp.where(x == lmax, col, _VB), axis=1, keepdims=True)
        return lmax, larg + j * _VB

    def _accum(lmax, larg):
        better = lmax > m_scr[...]
        m_scr[...] = jnp.where(better, lmax, m_scr[...])
        a_scr[...] = jnp.where(better, larg, a_scr[...])

    @pl.when(j == 0)
    def _init():
        lmax, larg = _reduce(x_ref[...])
        m_scr[...] = lmax
        a_scr[...] = larg

    @pl.when((j > 0) & (j < _NB - 1))
    def _mid():
        _accum(*_reduce(x_ref[...]))

    @pl.when(j == _NB - 1)
    def _fin():
        col = jax.lax.broadcasted_iota(jnp.int32, (_TC_ROWS, _VB), 1)
        x = jnp.where(col < _TAIL, x_ref[...], -jnp.inf)
        _accum(*_reduce(x))
        tt_ref[...] = a_scr[...]
        val_ref[...] = m_scr[...]


def _tc_argmax(logits):
    return pl.pallas_call(
        _tc_argmax_body,
        grid=(_NB,),
        in_specs=[pl.BlockSpec((_TC_ROWS, _VB), lambda j: (0, j))],
        out_specs=[
            pl.BlockSpec((_TC_ROWS, 1), lambda j: (0, 0)),
            pl.BlockSpec((_TC_ROWS, 1), lambda j: (0, 0)),
        ],
        out_shape=[
            jax.ShapeDtypeStruct((_TC_ROWS, 1), jnp.int32),
            jax.ShapeDtypeStruct((_TC_ROWS, 1), jnp.float32),
        ],
        scratch_shapes=[
            pltpu.VMEM((_TC_ROWS, 1), jnp.float32),
            pltpu.VMEM((_TC_ROWS, 1), jnp.int32),
        ],
    )(logits)


_NW = 32
_LANES = 16
_SPAN0 = 50048
_SPAN_END = 99968
_CW = 6272
_NFULL = 7
_TAIL0 = _SPAN0 - _NFULL * _CW
_TAIL1 = _SPAN_END - _SPAN0 - _NFULL * _CW


def _sc_argmax_body(logits_hbm, tt_out, val_out, buf, mref, aref, iref, vref,
                    sem0, sem1):
    cid = lax.axis_index("c")
    sid = lax.axis_index("s")
    wid = sid * 2 + cid
    grp = wid // 2
    span = wid % 2
    row0 = _TC_ROWS + grp * 8
    col0 = span * _SPAN0
    viota = lax.broadcasted_iota(jnp.int32, (_LANES,), 0)

    bufs = (buf.at[0], buf.at[1])
    sems = (sem0, sem1)

    def _issue(c, w):
        dst = bufs[c % 2] if w == _CW else bufs[c % 2].at[:, pl.ds(0, w)]
        pltpu.async_copy(
            logits_hbm.at[pl.ds(row0, 8), pl.ds(col0 + c * _CW, w)],
            dst, sems[c % 2])

    def _wait(c, w):
        dst = bufs[c % 2] if w == _CW else bufs[c % 2].at[:, pl.ds(0, w)]
        pltpu.make_async_copy(
            logits_hbm.at[pl.ds(row0, 8), pl.ds(0, w)], dst,
            sems[c % 2]).wait()

    _issue(0, _CW)

    for r in range(8):
        mref[r, :] = jnp.full((_LANES,), -jnp.inf, dtype=jnp.float32)
        aref[r, :] = jnp.zeros((_LANES,), dtype=jnp.int32)

    for c in range(_NFULL + 1):
        is_tail = c == _NFULL
        if not is_tail:
            if c + 1 < _NFULL:
                _issue(c + 1, _CW)
            else:
                @pl.when(span == 0)
                def _t0():
                    _issue(_NFULL, _TAIL0)

                @pl.when(span == 1)
                def _t1():
                    _issue(_NFULL, _TAIL1)

        b = bufs[c % 2]
        if is_tail:
            @pl.when(span == 0)
            def _w0():
                _wait(c, _TAIL0)

            @pl.when(span == 1)
            def _w1():
                _wait(c, _TAIL1)
        else:
            _wait(c, _CW)

        n_iters = (_TAIL1 // _LANES) if is_tail else (_CW // _LANES)

        for r in range(8):
            vbase = viota + (col0 + c * _CW)

            def inner(i, mc, b=b, r=r):
                vmax, varg, vcur = mc
                v = b[r, pl.ds(i * _LANES, _LANES)]
                take = v > vmax
                return (jnp.where(take, v, vmax),
                        jnp.where(take, vcur, varg),
                        vcur + _LANES)

            m1, a1, vc1 = lax.fori_loop(
                0, n_iters, inner, (mref[r, :], aref[r, :], vbase), unroll=4)
            if is_tail:
                @pl.when(span == 0)
                def _extra(inner=inner, m1=m1, a1=a1, vc1=vc1, r=r):
                    m2, a2, _ = lax.fori_loop(
                        _TAIL1 // _LANES, _TAIL0 // _LANES, inner,
                        (m1, a1, vc1))
                    mref[r, :] = m2
                    aref[r, :] = a2

                @pl.when(span == 1)
                def _noextra(m1=m1, a1=a1, r=r):
                    mref[r, :] = m1
                    aref[r, :] = a1
            else:
                mref[r, :] = m1
                aref[r, :] = a1

    vref[...] = jnp.zeros((_LANES,), dtype=jnp.float32)
    iref[...] = jnp.zeros((_LANES,), dtype=jnp.int32)
    for r in range(8):
        m = mref[r, :]
        a = aref[r, :]
        best = jnp.max(m)
        bidx = jnp.min(jnp.where(m == best, a, _VOCAB))
        lane = viota == r
        vref[...] = jnp.where(lane, best, vref[...])
        iref[...] = jnp.where(lane, bidx, iref[...])

    off = span * _SC_ROWS + grp * 8
    pltpu.sync_copy(iref.at[pl.ds(0, 8)], tt_out.at[pl.ds(off, 8)])
    pltpu.sync_copy(vref.at[pl.ds(0, 8)], val_out.at[pl.ds(off, 8)])


@functools.partial(
    pl.kernel,
    out_type=[
        jax.ShapeDtypeStruct((2 * _SC_ROWS,), jnp.int32),
        jax.ShapeDtypeStruct((2 * _SC_ROWS,), jnp.float32),
    ],
    mesh=plsc.VectorSubcoreMesh(core_axis_name="c", subcore_axis_name="s"),
    compiler_params=pltpu.CompilerParams(needs_layout_passes=False),
    scratch_types=[
        pltpu.VMEM((2, 8, _CW), jnp.float32),
        pltpu.VMEM((8, _LANES), jnp.float32),
        pltpu.VMEM((8, _LANES), jnp.int32),
        pltpu.VMEM((_LANES,), jnp.int32),
        pltpu.VMEM((_LANES,), jnp.float32),
        pltpu.SemaphoreType.DMA,
        pltpu.SemaphoreType.DMA,
    ],
)
def _sc_argmax(logits_hbm, tt_out, val_out, buf, mref, aref, iref, vref,
               sem0, sem1):
    _sc_argmax_body(logits_hbm, tt_out, val_out, buf, mref, aref, iref, vref,
                    sem0, sem1)


def _merge_accept_body(strip_ref, tt_tc_ref, val_tc_ref, tt_sc_ref,
                       val_sc_ref, draft_ref,
                       tok_ref, num_ref, val_ref):
    scol = jax.lax.broadcasted_iota(jnp.int32, (_ROWS, 128), 1)
    xs = jnp.where(scol < _VOCAB - _SPAN_END, strip_ref[...], -jnp.inf)
    sv_all = jnp.max(xs, axis=1, keepdims=True)
    st_all = jnp.min(jnp.where(xs == sv_all, scol, 128),
                     axis=1, keepdims=True) + _SPAN_END
    sv_col = sv_all[_TC_ROWS:]
    st_col = st_all[_TC_ROWS:]

    _g4 = (_NUM_GENS - _SC_GEN0, _MAX_DRAFT + 1)
    t_all = tt_sc_ref[...][:, 0]
    v_all = val_sc_ref[...][:, 0]
    t0 = t_all[:_SC_ROWS].reshape(_g4)
    t1 = t_all[_SC_ROWS:].reshape(_g4)
    v0 = v_all[:_SC_ROWS].reshape(_g4)
    v1 = v_all[_SC_ROWS:].reshape(_g4)
    b1 = v1 > v0
    mt = jnp.where(b1, t1, t0)
    mv = jnp.where(b1, v1, v0)
    st = st_col[:, 0].reshape(_g4)
    sv = sv_col[:, 0].reshape(_g4)
    b2 = sv > mv
    sc_t = jnp.where(b2, st, mt)
    sc_v = jnp.where(b2, sv, mv)

    tc_t = tt_tc_ref[...]
    tc_flat = tc_t[:, 0]
    gen_tc = tc_flat[_NUM_CONTEXTS:].reshape(_SC_GEN0, _MAX_DRAFT + 1)
    gen_t = jnp.concatenate([gen_tc, sc_t], axis=0)
    ctx = jnp.concatenate(
        [tc_t[:_NUM_CONTEXTS],
         jnp.zeros((_NUM_CONTEXTS, _MAX_DRAFT), dtype=jnp.int32)], axis=1)
    tok_ref[...] = jnp.concatenate([ctx, gen_t], axis=0)

    val_tc4 = val_tc_ref[...][:, 0].reshape(_TC_ROWS // 4, _MAX_DRAFT + 1)
    val_ref[...] = jnp.concatenate([val_tc4, sc_v], axis=0)

    draft = draft_ref[...]
    m = (draft == gen_t[:, :_MAX_DRAFT]).astype(jnp.int32)
    run = m[:, 0:1]
    total = run
    for k in range(1, _MAX_DRAFT):
        run = run * m[:, k:k + 1]
        total = total + run
    num_ref[...] = jnp.concatenate(
        [jnp.ones((_NUM_CONTEXTS, 1), jnp.int32), 1 + total], axis=0)


def _merge_accept(logits, tt_tc, val_tc, tt_sc, val_sc, draft):
    return pl.pallas_call(
        _merge_accept_body,
        grid=(1,),
        in_specs=[
            pl.BlockSpec((_ROWS, 128), lambda i: (0, _SPAN_END // 128)),
            pl.BlockSpec((_TC_ROWS, 1), lambda i: (0, 0)),
            pl.BlockSpec((_TC_ROWS, 1), lambda i: (0, 0)),
            pl.BlockSpec((2 * _SC_ROWS, 1), lambda i: (0, 0)),
            pl.BlockSpec((2 * _SC_ROWS, 1), lambda i: (0, 0)),
            pl.BlockSpec((_NUM_GENS, _MAX_DRAFT), lambda i: (0, 0)),
        ],
        out_specs=[
            pl.BlockSpec((_NUM_CONTEXTS + _NUM_GENS, _MAX_DRAFT + 1),
                         lambda i: (0, 0)),
            pl.BlockSpec((_NUM_CONTEXTS + _NUM_GENS, 1), lambda i: (0, 0)),
            pl.BlockSpec((_ROWS // 4, _MAX_DRAFT + 1), lambda i: (0, 0)),
        ],
        out_shape=[
            jax.ShapeDtypeStruct((_NUM_CONTEXTS + _NUM_GENS, _MAX_DRAFT + 1),
                                 jnp.int32),
            jax.ShapeDtypeStruct((_NUM_CONTEXTS + _NUM_GENS, 1), jnp.int32),
            jax.ShapeDtypeStruct((_ROWS // 4, _MAX_DRAFT + 1), jnp.float32),
        ],
    )(logits, tt_tc, val_tc, tt_sc, val_sc, draft)


@jax.jit
def kernel(logits, draft_tokens):
    if logits.ndim == 1:
        logits = logits[None, :]
    draft_tokens = draft_tokens.astype(jnp.int32)

    tt_tc, val_tc = _tc_argmax(logits)
    tt_sc, val_sc = _sc_argmax(logits)

    accepted_tokens, num_accepted, accepted_values = _merge_accept(
        logits, tt_tc, val_tc, tt_sc.reshape(2 * _SC_ROWS, 1),
        val_sc.reshape(2 * _SC_ROWS, 1), draft_tokens)
    return (accepted_tokens, num_accepted[:, 0],
            accepted_values.reshape(_ROWS))


# speedup vs baseline: 1.0211x; 1.0211x over previous
import functools

import jax
import jax.numpy as jnp
from jax import lax
from jax.experimental import pallas as pl
from jax.experimental.pallas import tpu as pltpu
from jax.experimental.pallas import tpu_sc as plsc

_NUM_CONTEXTS = 32
_NUM_GENS = 96
_MAX_DRAFT = 3
_ROWS = _NUM_CONTEXTS + _NUM_GENS * (_MAX_DRAFT + 1)
_VOCAB = 100000

_TC_ROWS = 288
_SC_ROWS = _ROWS - _TC_ROWS
_SC_GROUPS = _SC_ROWS // 8
_SC_GEN0 = (_TC_ROWS - _NUM_CONTEXTS) // 4

_VB = 8192
_NB = -(-_VOCAB // _VB)
_TAIL = _VOCAB - (_NB - 1) * _VB


def _tc_argmax_body(x_ref, tt_ref, val_ref, m_scr, a_scr):
    j = pl.program_id(0)

    def _reduce(x):
        col = jax.lax.broadcasted_iota(jnp.int32, (_TC_ROWS, _VB), 1)
        lmax = jnp.max(x, axis=1, keepdims=True)
        larg = jnp.min(jnp.where(x == lmax, col, _VB), axis=1, keepdims=True)
        return lmax, larg + j * _VB

    def _accum(lmax, larg):
        better = lmax > m_scr[...]
        m_scr[...] = jnp.where(better, lmax, m_scr[...])
        a_scr[...] = jnp.where(better, larg, a_scr[...])

    @pl.when(j == 0)
    def _init():
        lmax, larg = _reduce(x_ref[...])
        m_scr[...] = lmax
        a_scr[...] = larg

    @pl.when((j > 0) & (j < _NB - 1))
    def _mid():
        _accum(*_reduce(x_ref[...]))

    @pl.when(j == _NB - 1)
    def _fin():
        col = jax.lax.broadcasted_iota(jnp.int32, (_TC_ROWS, _VB), 1)
        x = jnp.where(col < _TAIL, x_ref[...], -jnp.inf)
        _accum(*_reduce(x))
        tt_ref[...] = a_scr[...]
        val_ref[...] = m_scr[...]


def _tc_argmax(logits):
    return pl.pallas_call(
        _tc_argmax_body,
        grid=(_NB,),
        in_specs=[pl.BlockSpec((_TC_ROWS, _VB), lambda j: (0, j))],
        out_specs=[
            pl.BlockSpec((_TC_ROWS, 1), lambda j: (0, 0)),
            pl.BlockSpec((_TC_ROWS, 1), lambda j: (0, 0)),
        ],
        out_shape=[
            jax.ShapeDtypeStruct((_TC_ROWS, 1), jnp.int32),
            jax.ShapeDtypeStruct((_TC_ROWS, 1), jnp.float32),
        ],
        scratch_shapes=[
            pltpu.VMEM((_TC_ROWS, 1), jnp.float32),
            pltpu.VMEM((_TC_ROWS, 1), jnp.int32),
        ],
    )(logits)


_NW = 32
_LANES = 16
_SPAN0 = 50048
_SPAN_END = 99968
_CW = 6272
_NFULL = 7
_TAIL0 = _SPAN0 - _NFULL * _CW
_TAIL1 = _SPAN_END - _SPAN0 - _NFULL * _CW


def _sc_argmax_body(logits_hbm, tt_out, val_out, buf, mref, aref, iref, vref,
                    sem0, sem1):
    cid = lax.axis_index("c")
    sid = lax.axis_index("s")
    wid = sid * 2 + cid
    grp = wid // 2
    span = wid % 2
    row0 = _TC_ROWS + grp * 8
    col0 = span * _SPAN0
    viota = lax.broadcasted_iota(jnp.int32, (_LANES,), 0)

    bufs = (buf.at[0], buf.at[1])
    sems = (sem0, sem1)

    def _issue(c, w):
        dst = bufs[c % 2] if w == _CW else bufs[c % 2].at[:, pl.ds(0, w)]
        pltpu.async_copy(
            logits_hbm.at[pl.ds(row0, 8), pl.ds(col0 + c * _CW, w)],
            dst, sems[c % 2])

    def _wait(c, w):
        dst = bufs[c % 2] if w == _CW else bufs[c % 2].at[:, pl.ds(0, w)]
        pltpu.make_async_copy(
            logits_hbm.at[pl.ds(row0, 8), pl.ds(0, w)], dst,
            sems[c % 2]).wait()

    _issue(0, _CW)

    for r in range(8):
        mref[r, :] = jnp.full((_LANES,), -jnp.inf, dtype=jnp.float32)
        aref[r, :] = jnp.zeros((_LANES,), dtype=jnp.int32)

    for c in range(_NFULL + 1):
        is_tail = c == _NFULL
        if not is_tail:
            if c + 1 < _NFULL:
                _issue(c + 1, _CW)
            else:
                @pl.when(span == 0)
                def _t0():
                    _issue(_NFULL, _TAIL0)

                @pl.when(span == 1)
                def _t1():
                    _issue(_NFULL, _TAIL1)

        b = bufs[c % 2]
        if is_tail:
            @pl.when(span == 0)
            def _w0():
                _wait(c, _TAIL0)

            @pl.when(span == 1)
            def _w1():
                _wait(c, _TAIL1)
        else:
            _wait(c, _CW)

        n_iters = (_TAIL1 // _LANES) if is_tail else (_CW // _LANES)

        for r in range(8):
            vbase = viota + (col0 + c * _CW)

            def inner(i, mc, b=b, r=r):
                vmax, varg, vcur = mc
                v = b[r, pl.ds(i * _LANES, _LANES)]
                take = v > vmax
                return (jnp.where(take, v, vmax),
                        jnp.where(take, vcur, varg),
                        vcur + _LANES)

            m1, a1, vc1 = lax.fori_loop(
                0, n_iters, inner, (mref[r, :], aref[r, :], vbase), unroll=4)
            if is_tail:
                @pl.when(span == 0)
                def _extra(inner=inner, m1=m1, a1=a1, vc1=vc1, r=r):
                    m2, a2, _ = lax.fori_loop(
                        _TAIL1 // _LANES, _TAIL0 // _LANES, inner,
                        (m1, a1, vc1))
                    mref[r, :] = m2
                    aref[r, :] = a2

                @pl.when(span == 1)
                def _noextra(m1=m1, a1=a1, r=r):
                    mref[r, :] = m1
                    aref[r, :] = a1
            else:
                mref[r, :] = m1
                aref[r, :] = a1

    vref[...] = jnp.zeros((_LANES,), dtype=jnp.float32)
    iref[...] = jnp.zeros((_LANES,), dtype=jnp.int32)
    for r in range(8):
        m = mref[r, :]
        a = aref[r, :]
        best = jnp.max(m)
        bidx = jnp.min(jnp.where(m == best, a, _VOCAB))
        lane = viota == r
        vref[...] = jnp.where(lane, best, vref[...])
        iref[...] = jnp.where(lane, bidx, iref[...])

    off = span * _SC_ROWS + grp * 8
    pltpu.sync_copy(iref.at[pl.ds(0, 8)], tt_out.at[pl.ds(off, 8)])
    pltpu.sync_copy(vref.at[pl.ds(0, 8)], val_out.at[pl.ds(off, 8)])


@functools.partial(
    pl.kernel,
    out_type=[
        jax.ShapeDtypeStruct((2 * _SC_ROWS,), jnp.int32),
        jax.ShapeDtypeStruct((2 * _SC_ROWS,), jnp.float32),
    ],
    mesh=plsc.VectorSubcoreMesh(core_axis_name="c", subcore_axis_name="s"),
    compiler_params=pltpu.CompilerParams(needs_layout_passes=False),
    scratch_types=[
        pltpu.VMEM((2, 8, _CW), jnp.float32),
        pltpu.VMEM((8, _LANES), jnp.float32),
        pltpu.VMEM((8, _LANES), jnp.int32),
        pltpu.VMEM((_LANES,), jnp.int32),
        pltpu.VMEM((_LANES,), jnp.float32),
        pltpu.SemaphoreType.DMA,
        pltpu.SemaphoreType.DMA,
    ],
)
def _sc_argmax(logits_hbm, tt_out, val_out, buf, mref, aref, iref, vref,
               sem0, sem1):
    _sc_argmax_body(logits_hbm, tt_out, val_out, buf, mref, aref, iref, vref,
                    sem0, sem1)


def _merge_accept_body(strip_ref, tt_tc_ref, val_tc_ref, tt_sc_ref,
                       val_sc_ref, draft_ref,
                       tok_ref, num_ref, val_ref):
    scol = jax.lax.broadcasted_iota(jnp.int32, (_ROWS, 128), 1)
    xs = jnp.where(scol < _VOCAB - _SPAN_END, strip_ref[...], -jnp.inf)
    sv_all = jnp.max(xs, axis=1, keepdims=True)
    st_all = jnp.min(jnp.where(xs == sv_all, scol, 128),
                     axis=1, keepdims=True) + _SPAN_END
    sv_col = sv_all[_TC_ROWS:]
    st_col = st_all[_TC_ROWS:]

    _g4 = (_NUM_GENS - _SC_GEN0, _MAX_DRAFT + 1)
    t_all = tt_sc_ref[...][:, 0]
    v_all = val_sc_ref[...][:, 0]
    t0 = t_all[:_SC_ROWS].reshape(_g4)
    t1 = t_all[_SC_ROWS:].reshape(_g4)
    v0 = v_all[:_SC_ROWS].reshape(_g4)
    v1 = v_all[_SC_ROWS:].reshape(_g4)
    b1 = v1 > v0
    mt = jnp.where(b1, t1, t0)
    mv = jnp.where(b1, v1, v0)
    st = st_col[:, 0].reshape(_g4)
    sv = sv_col[:, 0].reshape(_g4)
    b2 = sv > mv
    sc_t = jnp.where(b2, st, mt)
    sc_v = jnp.where(b2, sv, mv)

    tc_t = tt_tc_ref[...]
    tc_flat = tc_t[:, 0]
    gen_tc = tc_flat[_NUM_CONTEXTS:].reshape(_SC_GEN0, _MAX_DRAFT + 1)
    gen_t = jnp.concatenate([gen_tc, sc_t], axis=0)
    ctx = jnp.concatenate(
        [tc_t[:_NUM_CONTEXTS],
         jnp.zeros((_NUM_CONTEXTS, _MAX_DRAFT), dtype=jnp.int32)], axis=1)
    tok_ref[...] = jnp.concatenate([ctx, gen_t], axis=0)

    val_tc4 = val_tc_ref[...][:, 0].reshape(_TC_ROWS // 4, _MAX_DRAFT + 1)
    val_ref[...] = jnp.concatenate([val_tc4, sc_v], axis=0)

    draft = draft_ref[...]
    m = (draft == gen_t[:, :_MAX_DRAFT]).astype(jnp.int32)
    run = m[:, 0:1]
    total = run
    for k in range(1, _MAX_DRAFT):
        run = run * m[:, k:k + 1]
        total = total + run
    num_ref[...] = jnp.concatenate(
        [jnp.ones((_NUM_CONTEXTS, 1), jnp.int32), 1 + total], axis=0)


def _merge_accept(logits, tt_tc, val_tc, tt_sc, val_sc, draft):
    return pl.pallas_call(
        _merge_accept_body,
        grid=(1,),
        in_specs=[
            pl.BlockSpec((_ROWS, 128), lambda i: (0, _SPAN_END // 128)),
            pl.BlockSpec((_TC_ROWS, 1), lambda i: (0, 0)),
            pl.BlockSpec((_TC_ROWS, 1), lambda i: (0, 0)),
            pl.BlockSpec((2 * _SC_ROWS, 1), lambda i: (0, 0)),
            pl.BlockSpec((2 * _SC_ROWS, 1), lambda i: (0, 0)),
            pl.BlockSpec((_NUM_GENS, _MAX_DRAFT), lambda i: (0, 0)),
        ],
        out_specs=[
            pl.BlockSpec((_NUM_CONTEXTS + _NUM_GENS, _MAX_DRAFT + 1),
                         lambda i: (0, 0)),
            pl.BlockSpec((_NUM_CONTEXTS + _NUM_GENS, 1), lambda i: (0, 0)),
            pl.BlockSpec((_ROWS // 4, _MAX_DRAFT + 1), lambda i: (0, 0)),
        ],
        out_shape=[
            jax.ShapeDtypeStruct((_NUM_CONTEXTS + _NUM_GENS, _MAX_DRAFT + 1),
                                 jnp.int32),
            jax.ShapeDtypeStruct((_NUM_CONTEXTS + _NUM_GENS, 1), jnp.int32),
            jax.ShapeDtypeStruct((_ROWS // 4, _MAX_DRAFT + 1), jnp.float32),
        ],
    )(logits, tt_tc, val_tc, tt_sc, val_sc, draft)


@jax.jit
def kernel(logits, draft_tokens):
    if logits.ndim == 1:
        logits = logits[None, :]
    draft_tokens = draft_tokens.astype(jnp.int32)

    tt_tc, val_tc = _tc_argmax(logits)
    tt_sc, val_sc = _sc_argmax(logits)

    accepted_tokens, num_accepted, accepted_values = _merge_accept(
        logits, tt_tc, val_tc, tt_sc.reshape(2 * _SC_ROWS, 1),
        val_sc.reshape(2 * _SC_ROWS, 1), draft_tokens)
    return (accepted_tokens, num_accepted[:, 0],
            accepted_values.reshape(_ROWS))


# speedup vs baseline: 1.0247x; 1.0035x over previous
import functools

import jax
import jax.numpy as jnp
from jax import lax
from jax.experimental import pallas as pl
from jax.experimental.pallas import tpu as pltpu
from jax.experimental.pallas import tpu_sc as plsc

_NUM_CONTEXTS = 32
_NUM_GENS = 96
_MAX_DRAFT = 3
_ROWS = _NUM_CONTEXTS + _NUM_GENS * (_MAX_DRAFT + 1)
_VOCAB = 100000

_TC_ROWS = 288
_SC_ROWS = _ROWS - _TC_ROWS
_SC_GROUPS = _SC_ROWS // 8
_SC_GEN0 = (_TC_ROWS - _NUM_CONTEXTS) // 4

_VB = 8192
_NB = -(-_VOCAB // _VB)
_TAIL = _VOCAB - (_NB - 1) * _VB


def _tc_argmax_body(x_ref, tt_ref, val_ref, m_scr, a_scr):
    j = pl.program_id(0)

    def _reduce(x):
        col = jax.lax.broadcasted_iota(jnp.int32, (_TC_ROWS, _VB), 1)
        lmax = jnp.max(x, axis=1, keepdims=True)
        larg = jnp.min(jnp.where(x == lmax, col, _VB), axis=1, keepdims=True)
        return lmax, larg + j * _VB

    def _accum(lmax, larg):
        better = lmax > m_scr[...]
        m_scr[...] = jnp.where(better, lmax, m_scr[...])
        a_scr[...] = jnp.where(better, larg, a_scr[...])

    @pl.when(j == 0)
    def _init():
        lmax, larg = _reduce(x_ref[...])
        m_scr[...] = lmax
        a_scr[...] = larg

    @pl.when((j > 0) & (j < _NB - 1))
    def _mid():
        _accum(*_reduce(x_ref[...]))

    @pl.when(j == _NB - 1)
    def _fin():
        col = jax.lax.broadcasted_iota(jnp.int32, (_TC_ROWS, _VB), 1)
        x = jnp.where(col < _TAIL, x_ref[...], -jnp.inf)
        _accum(*_reduce(x))
        tt_ref[...] = a_scr[...]
        val_ref[...] = m_scr[...]


def _tc_argmax(logits):
    return pl.pallas_call(
        _tc_argmax_body,
        grid=(_NB,),
        in_specs=[pl.BlockSpec((_TC_ROWS, _VB), lambda j: (0, j))],
        out_specs=[
            pl.BlockSpec((_TC_ROWS, 1), lambda j: (0, 0)),
            pl.BlockSpec((_TC_ROWS, 1), lambda j: (0, 0)),
        ],
        out_shape=[
            jax.ShapeDtypeStruct((_TC_ROWS, 1), jnp.int32),
            jax.ShapeDtypeStruct((_TC_ROWS, 1), jnp.float32),
        ],
        scratch_shapes=[
            pltpu.VMEM((_TC_ROWS, 1), jnp.float32),
            pltpu.VMEM((_TC_ROWS, 1), jnp.int32),
        ],
    )(logits)


_NW = 32
_LANES = 16
_SPAN0 = 50048
_SPAN_END = 99968
_CW = 6272
_NFULL = 7
_TAIL0 = _SPAN0 - _NFULL * _CW
_TAIL1 = _SPAN_END - _SPAN0 - _NFULL * _CW


def _sc_argmax_body(logits_hbm, tt_out, val_out, buf, mref, aref, iref, vref,
                    sem0, sem1):
    cid = lax.axis_index("c")
    sid = lax.axis_index("s")
    wid = sid * 2 + cid
    grp = wid // 2
    span = wid % 2
    row0 = _TC_ROWS + grp * 8
    col0 = span * _SPAN0
    viota = lax.broadcasted_iota(jnp.int32, (_LANES,), 0)

    bufs = (buf.at[0], buf.at[1])
    sems = (sem0, sem1)

    def _issue(c, w):
        dst = bufs[c % 2] if w == _CW else bufs[c % 2].at[:, pl.ds(0, w)]
        pltpu.async_copy(
            logits_hbm.at[pl.ds(row0, 8), pl.ds(col0 + c * _CW, w)],
            dst, sems[c % 2])

    def _wait(c, w):
        dst = bufs[c % 2] if w == _CW else bufs[c % 2].at[:, pl.ds(0, w)]
        pltpu.make_async_copy(
            logits_hbm.at[pl.ds(row0, 8), pl.ds(0, w)], dst,
            sems[c % 2]).wait()

    _issue(0, _CW)

    for r in range(8):
        mref[r, :] = jnp.full((_LANES,), -jnp.inf, dtype=jnp.float32)
        aref[r, :] = jnp.zeros((_LANES,), dtype=jnp.int32)

    for c in range(_NFULL + 1):
        is_tail = c == _NFULL
        if not is_tail:
            if c + 1 < _NFULL:
                _issue(c + 1, _CW)
            else:
                @pl.when(span == 0)
                def _t0():
                    _issue(_NFULL, _TAIL0)

                @pl.when(span == 1)
                def _t1():
                    _issue(_NFULL, _TAIL1)

        b = bufs[c % 2]
        if is_tail:
            @pl.when(span == 0)
            def _w0():
                _wait(c, _TAIL0)

            @pl.when(span == 1)
            def _w1():
                _wait(c, _TAIL1)
        else:
            _wait(c, _CW)

        n_iters = (_TAIL1 // _LANES) if is_tail else (_CW // _LANES)

        for r in range(8):
            vbase = viota + (col0 + c * _CW)

            def inner(i, mc, b=b, r=r):
                vmax, varg, vcur = mc
                v = b[r, pl.ds(i * _LANES, _LANES)]
                take = v > vmax
                return (jnp.where(take, v, vmax),
                        jnp.where(take, vcur, varg),
                        vcur + _LANES)

            m1, a1, vc1 = lax.fori_loop(
                0, n_iters, inner, (mref[r, :], aref[r, :], vbase), unroll=8)
            if is_tail:
                @pl.when(span == 0)
                def _extra(inner=inner, m1=m1, a1=a1, vc1=vc1, r=r):
                    m2, a2, _ = lax.fori_loop(
                        _TAIL1 // _LANES, _TAIL0 // _LANES, inner,
                        (m1, a1, vc1))
                    mref[r, :] = m2
                    aref[r, :] = a2

                @pl.when(span == 1)
                def _noextra(m1=m1, a1=a1, r=r):
                    mref[r, :] = m1
                    aref[r, :] = a1
            else:
                mref[r, :] = m1
                aref[r, :] = a1

    vref[...] = jnp.zeros((_LANES,), dtype=jnp.float32)
    iref[...] = jnp.zeros((_LANES,), dtype=jnp.int32)
    for r in range(8):
        m = mref[r, :]
        a = aref[r, :]
        best = jnp.max(m)
        bidx = jnp.min(jnp.where(m == best, a, _VOCAB))
        lane = viota == r
        vref[...] = jnp.where(lane, best, vref[...])
        iref[...] = jnp.where(lane, bidx, iref[...])

    off = span * _SC_ROWS + grp * 8
    pltpu.sync_copy(iref.at[pl.ds(0, 8)], tt_out.at[pl.ds(off, 8)])
    pltpu.sync_copy(vref.at[pl.ds(0, 8)], val_out.at[pl.ds(off, 8)])


@functools.partial(
    pl.kernel,
    out_type=[
        jax.ShapeDtypeStruct((2 * _SC_ROWS,), jnp.int32),
        jax.ShapeDtypeStruct((2 * _SC_ROWS,), jnp.float32),
    ],
    mesh=plsc.VectorSubcoreMesh(core_axis_name="c", subcore_axis_name="s"),
    compiler_params=pltpu.CompilerParams(needs_layout_passes=False,
                                         skip_device_barrier=True),
    scratch_types=[
        pltpu.VMEM((2, 8, _CW), jnp.float32),
        pltpu.VMEM((8, _LANES), jnp.float32),
        pltpu.VMEM((8, _LANES), jnp.int32),
        pltpu.VMEM((_LANES,), jnp.int32),
        pltpu.VMEM((_LANES,), jnp.float32),
        pltpu.SemaphoreType.DMA,
        pltpu.SemaphoreType.DMA,
    ],
)
def _sc_argmax(logits_hbm, tt_out, val_out, buf, mref, aref, iref, vref,
               sem0, sem1):
    _sc_argmax_body(logits_hbm, tt_out, val_out, buf, mref, aref, iref, vref,
                    sem0, sem1)


def _merge_accept_body(strip_ref, tt_tc_ref, val_tc_ref, tt_sc_ref,
                       val_sc_ref, draft_ref,
                       tok_ref, num_ref, val_ref):
    scol = jax.lax.broadcasted_iota(jnp.int32, (_ROWS, 128), 1)
    xs = jnp.where(scol < _VOCAB - _SPAN_END, strip_ref[...], -jnp.inf)
    sv_all = jnp.max(xs, axis=1, keepdims=True)
    st_all = jnp.min(jnp.where(xs == sv_all, scol, 128),
                     axis=1, keepdims=True) + _SPAN_END
    sv_col = sv_all[_TC_ROWS:]
    st_col = st_all[_TC_ROWS:]

    _g4 = (_NUM_GENS - _SC_GEN0, _MAX_DRAFT + 1)
    t_all = tt_sc_ref[...][:, 0]
    v_all = val_sc_ref[...][:, 0]
    t0 = t_all[:_SC_ROWS].reshape(_g4)
    t1 = t_all[_SC_ROWS:].reshape(_g4)
    v0 = v_all[:_SC_ROWS].reshape(_g4)
    v1 = v_all[_SC_ROWS:].reshape(_g4)
    b1 = v1 > v0
    mt = jnp.where(b1, t1, t0)
    mv = jnp.where(b1, v1, v0)
    st = st_col[:, 0].reshape(_g4)
    sv = sv_col[:, 0].reshape(_g4)
    b2 = sv > mv
    sc_t = jnp.where(b2, st, mt)
    sc_v = jnp.where(b2, sv, mv)

    tc_t = tt_tc_ref[...]
    tc_flat = tc_t[:, 0]
    gen_tc = tc_flat[_NUM_CONTEXTS:].reshape(_SC_GEN0, _MAX_DRAFT + 1)
    gen_t = jnp.concatenate([gen_tc, sc_t], axis=0)
    ctx = jnp.concatenate(
        [tc_t[:_NUM_CONTEXTS],
         jnp.zeros((_NUM_CONTEXTS, _MAX_DRAFT), dtype=jnp.int32)], axis=1)
    tok_ref[...] = jnp.concatenate([ctx, gen_t], axis=0)

    val_tc4 = val_tc_ref[...][:, 0].reshape(_TC_ROWS // 4, _MAX_DRAFT + 1)
    val_ref[...] = jnp.concatenate([val_tc4, sc_v], axis=0)

    draft = draft_ref[...]
    m = (draft == gen_t[:, :_MAX_DRAFT]).astype(jnp.int32)
    run = m[:, 0:1]
    total = run
    for k in range(1, _MAX_DRAFT):
        run = run * m[:, k:k + 1]
        total = total + run
    num_ref[...] = jnp.concatenate(
        [jnp.ones((_NUM_CONTEXTS, 1), jnp.int32), 1 + total], axis=0)


def _merge_accept(logits, tt_tc, val_tc, tt_sc, val_sc, draft):
    return pl.pallas_call(
        _merge_accept_body,
        grid=(1,),
        in_specs=[
            pl.BlockSpec((_ROWS, 128), lambda i: (0, _SPAN_END // 128)),
            pl.BlockSpec((_TC_ROWS, 1), lambda i: (0, 0)),
            pl.BlockSpec((_TC_ROWS, 1), lambda i: (0, 0)),
            pl.BlockSpec((2 * _SC_ROWS, 1), lambda i: (0, 0)),
            pl.BlockSpec((2 * _SC_ROWS, 1), lambda i: (0, 0)),
            pl.BlockSpec((_NUM_GENS, _MAX_DRAFT), lambda i: (0, 0)),
        ],
        out_specs=[
            pl.BlockSpec((_NUM_CONTEXTS + _NUM_GENS, _MAX_DRAFT + 1),
                         lambda i: (0, 0)),
            pl.BlockSpec((_NUM_CONTEXTS + _NUM_GENS, 1), lambda i: (0, 0)),
            pl.BlockSpec((_ROWS // 4, _MAX_DRAFT + 1), lambda i: (0, 0)),
        ],
        out_shape=[
            jax.ShapeDtypeStruct((_NUM_CONTEXTS + _NUM_GENS, _MAX_DRAFT + 1),
                                 jnp.int32),
            jax.ShapeDtypeStruct((_NUM_CONTEXTS + _NUM_GENS, 1), jnp.int32),
            jax.ShapeDtypeStruct((_ROWS // 4, _MAX_DRAFT + 1), jnp.float32),
        ],
    )(logits, tt_tc, val_tc, tt_sc, val_sc, draft)


@jax.jit
def kernel(logits, draft_tokens):
    if logits.ndim == 1:
        logits = logits[None, :]
    draft_tokens = draft_tokens.astype(jnp.int32)

    tt_tc, val_tc = _tc_argmax(logits)
    tt_sc, val_sc = _sc_argmax(logits)

    accepted_tokens, num_accepted, accepted_values = _merge_accept(
        logits, tt_tc, val_tc, tt_sc.reshape(2 * _SC_ROWS, 1),
        val_sc.reshape(2 * _SC_ROWS, 1), draft_tokens)
    return (accepted_tokens, num_accepted[:, 0],
            accepted_values.reshape(_ROWS))
